# baseline (speedup 1.0000x reference)
import jax
from jax.experimental import pallas as pl
from jax.experimental.pallas import tpu as pltpu


_CHUNK_ROWS = 5000
_RING = 4


def _ring_copy_kernel(u, i, t, ou, oi, ot, b0, b1, b2, b3,
                      l0, l1, l2, l3, s0, s1, s2, s3):
    bufs = (b0, b1, b2, b3)
    lsems = (l0, l1, l2, l3)
    ssems = (s0, s1, s2, s3)

    chunks = []
    for src, dst in ((u, ou), (i, oi), (t, ot)):
        n = src.shape[0] // _CHUNK_ROWS
        for k in range(n):
            sl = pl.ds(k * _CHUNK_ROWS, _CHUNK_ROWS)
            chunks.append((src.at[sl], dst.at[sl]))

    n_chunks = len(chunks)
    load_h = [None] * _RING
    store_h = [None] * _RING

    def start_load(j):
        b = j % _RING
        h = pltpu.make_async_copy(chunks[j][0], bufs[b], lsems[b])
        h.start()
        load_h[b] = h

    for j in range(min(_RING - 1, n_chunks)):
        start_load(j)

    for idx in range(n_chunks):
        b = idx % _RING
        load_h[b].wait()
        h = pltpu.make_async_copy(bufs[b], chunks[idx][1], ssems[b])
        h.start()
        store_h[b] = h
        j = idx + _RING - 1
        if j < n_chunks:
            jb = j % _RING
            if store_h[jb] is not None:
                store_h[jb].wait()
                store_h[jb] = None
            start_load(j)

    for h in store_h:
        if h is not None:
            h.wait()


def kernel(embed_user, embed_item, embed_tag):
    d = embed_user.shape[1]
    hbm_spec = pl.BlockSpec(memory_space=pltpu.MemorySpace.HBM)
    buf = pltpu.VMEM((_CHUNK_ROWS, d), embed_user.dtype)
    return pl.pallas_call(
        _ring_copy_kernel,
        in_specs=[hbm_spec] * 3,
        out_specs=[hbm_spec] * 3,
        scratch_shapes=[buf] * _RING + [pltpu.SemaphoreType.DMA] * (2 * _RING),
        out_shape=[
            jax.ShapeDtypeStruct(embed_user.shape, embed_user.dtype),
            jax.ShapeDtypeStruct(embed_item.shape, embed_item.dtype),
            jax.ShapeDtypeStruct(embed_tag.shape, embed_tag.dtype),
        ],
    )(embed_user, embed_item, embed_tag)


# speedup vs baseline: 1.0030x; 1.0030x over previous
import jax
from jax.experimental import pallas as pl
from jax.experimental.pallas import tpu as pltpu


_CHUNK_ROWS = 5000
_RING = 8
_LOOKAHEAD = _RING - 2


def _ring_copy_kernel(u, i, t, ou, oi, ot, *scratch):
    bufs = scratch[:_RING]
    lsems = scratch[_RING:2 * _RING]
    ssems = scratch[2 * _RING:]

    chunks = []
    for src, dst in ((u, ou), (i, oi), (t, ot)):
        n = src.shape[0] // _CHUNK_ROWS
        for k in range(n):
            sl = pl.ds(k * _CHUNK_ROWS, _CHUNK_ROWS)
            chunks.append((src.at[sl], dst.at[sl]))

    n_chunks = len(chunks)
    load_h = [None] * _RING
    store_h = [None] * _RING

    def start_load(j):
        b = j % _RING
        h = pltpu.make_async_copy(chunks[j][0], bufs[b], lsems[b])
        h.start()
        load_h[b] = h

    for j in range(min(_LOOKAHEAD, n_chunks)):
        start_load(j)

    for idx in range(n_chunks):
        b = idx % _RING
        load_h[b].wait()
        h = pltpu.make_async_copy(bufs[b], chunks[idx][1], ssems[b])
        h.start()
        store_h[b] = h
        j = idx + _LOOKAHEAD
        if j < n_chunks:
            jb = j % _RING
            if store_h[jb] is not None:
                store_h[jb].wait()
                store_h[jb] = None
            start_load(j)

    for h in store_h:
        if h is not None:
            h.wait()


def kernel(embed_user, embed_item, embed_tag):
    d = embed_user.shape[1]
    hbm_spec = pl.BlockSpec(memory_space=pltpu.MemorySpace.HBM)
    buf = pltpu.VMEM((_CHUNK_ROWS, d), embed_user.dtype)
    return pl.pallas_call(
        _ring_copy_kernel,
        in_specs=[hbm_spec] * 3,
        out_specs=[hbm_spec] * 3,
        scratch_shapes=[buf] * _RING + [pltpu.SemaphoreType.DMA] * (2 * _RING),
        out_shape=[
            jax.ShapeDtypeStruct(embed_user.shape, embed_user.dtype),
            jax.ShapeDtypeStruct(embed_item.shape, embed_item.dtype),
            jax.ShapeDtypeStruct(embed_tag.shape, embed_tag.dtype),
        ],
    )(embed_user, embed_item, embed_tag)


# speedup vs baseline: 1.0111x; 1.0081x over previous
import jax
from jax.experimental import pallas as pl
from jax.experimental.pallas import tpu as pltpu


_TC_STEPS = 10


def _copy3_kernel(u_ref, i_ref, t_ref, ou_ref, oi_ref, ot_ref):
    ou_ref[...] = u_ref[...]
    oi_ref[...] = i_ref[...]
    ot_ref[...] = t_ref[...]


def kernel(embed_user, embed_item, embed_tag):
    nu, d = embed_user.shape
    ni, _ = embed_item.shape
    nt, _ = embed_tag.shape
    bu, bi, bt = nu // _TC_STEPS, ni // _TC_STEPS, nt // _TC_STEPS

    def spec(block_rows):
        return pl.BlockSpec((block_rows, d), lambda s: (s, 0))

    return pl.pallas_call(
        _copy3_kernel,
        grid=(_TC_STEPS,),
        compiler_params=pltpu.CompilerParams(dimension_semantics=("parallel",)),
        in_specs=[spec(bu), spec(bi), spec(bt)],
        out_specs=[spec(bu), spec(bi), spec(bt)],
        out_shape=[
            jax.ShapeDtypeStruct(embed_user.shape, embed_user.dtype),
            jax.ShapeDtypeStruct(embed_item.shape, embed_item.dtype),
            jax.ShapeDtypeStruct(embed_tag.shape, embed_tag.dtype),
        ],
    )(embed_user, embed_item, embed_tag)
